# Initial kernel scaffold; baseline (speedup 1.0000x reference)
import functools

import jax
import jax.numpy as jnp
from jax import lax
from jax.experimental import pallas as pl
from jax.experimental.pallas import tpu as pltpu
from jax.experimental.pallas import tpu_sc as plsc

NUM_CORES = 2
NUM_SUBCORES = 16
NUM_WORKERS = NUM_CORES * NUM_SUBCORES
CHUNK = 128


def kernel(indices, table):
    batch, n_fields = indices.shape
    n_rows, dim = table.shape
    total = batch * n_fields
    assert total % (NUM_WORKERS * CHUNK) == 0
    per_worker = total // NUM_WORKERS
    n_chunks = per_worker // CHUNK

    idx3 = indices.reshape(NUM_WORKERS, n_chunks, CHUNK)
    mesh = plsc.VectorSubcoreMesh(core_axis_name="c", subcore_axis_name="s")

    @functools.partial(
        pl.kernel,
        mesh=mesh,
        out_type=jax.ShapeDtypeStruct((total, dim), jnp.float32),
        scratch_types=[
            pltpu.VMEM((n_chunks, CHUNK), jnp.int32),
            pltpu.VMEM((CHUNK, dim), jnp.float32),
            pltpu.SemaphoreType.DMA,
        ],
    )
    def gather_kernel(idx_hbm, table_hbm, out_hbm, idx_v, rows_v, sem):
        wid = lax.axis_index("s") * NUM_CORES + lax.axis_index("c")
        base = wid * per_worker
        pltpu.sync_copy(idx_hbm.at[wid], idx_v)

        def step(j, carry):
            pltpu.async_copy(table_hbm.at[idx_v.at[j]], rows_v, sem).wait()
            pltpu.sync_copy(rows_v, out_hbm.at[pl.ds(base + j * CHUNK, CHUNK)])
            return carry

        lax.fori_loop(0, n_chunks, step, 0)

    out = gather_kernel(idx3, table)
    return out.reshape(batch, n_fields, dim)


# speedup vs baseline: 1.5750x; 1.5750x over previous
import functools

import jax
import jax.numpy as jnp
from jax import lax
from jax.experimental import pallas as pl
from jax.experimental.pallas import tpu as pltpu
from jax.experimental.pallas import tpu_sc as plsc

NUM_CORES = 2
NUM_SUBCORES = 16
NUM_WORKERS = NUM_CORES * NUM_SUBCORES
CHUNK = 128
K = 13
GROUP = CHUNK * K


def kernel(indices, table):
    batch, n_fields = indices.shape
    n_rows, dim = table.shape
    total = batch * n_fields
    assert total % (NUM_WORKERS * GROUP) == 0
    per_worker = total // NUM_WORKERS
    n_chunks = per_worker // CHUNK
    n_groups = per_worker // GROUP
    assert n_groups % 2 == 0

    idx3 = indices.reshape(NUM_WORKERS, n_chunks, CHUNK)
    mesh = plsc.VectorSubcoreMesh(core_axis_name="c", subcore_axis_name="s")

    @functools.partial(
        pl.kernel,
        mesh=mesh,
        out_type=jax.ShapeDtypeStruct((total, dim), jnp.float32),
        scratch_types=[
            pltpu.VMEM((n_chunks, CHUNK), jnp.int32),
            pltpu.VMEM((GROUP, dim), jnp.float32),
            pltpu.VMEM((GROUP, dim), jnp.float32),
            pltpu.SemaphoreType.DMA,
            pltpu.SemaphoreType.DMA,
            pltpu.SemaphoreType.DMA,
            pltpu.SemaphoreType.DMA,
        ],
        compiler_params=pltpu.CompilerParams(use_tc_tiling_on_sc=False),
    )
    def gather_kernel(idx_hbm, table_hbm, out_hbm, idx_v, buf0, buf1,
                      gsem0, gsem1, ssem0, ssem1):
        wid = lax.axis_index("s") * NUM_CORES + lax.axis_index("c")
        base = wid * per_worker
        pltpu.sync_copy(idx_hbm.at[wid], idx_v)

        def fire_group(g, buf, gsem):
            for j in range(K):
                pltpu.async_copy(
                    table_hbm.at[idx_v.at[g * K + j]],
                    buf.at[pl.ds(j * CHUNK, CHUNK)],
                    gsem,
                )

        def drain_group(buf, gsem):
            pltpu.make_async_copy(
                table_hbm.at[pl.ds(0, GROUP)], buf, gsem
            ).wait()

        def store_group(g, buf, ssem):
            return pltpu.async_copy(
                buf, out_hbm.at[pl.ds(base + g * GROUP, GROUP)], ssem
            )

        def drain_store(g, buf, ssem):
            pltpu.make_async_copy(
                buf, out_hbm.at[pl.ds(base + g * GROUP, GROUP)], ssem
            ).wait()

        fire_group(0, buf0, gsem0)

        @pl.loop(0, n_groups, step=2)
        def _(i2):
            g0 = i2
            g1 = i2 + 1

            @pl.when(g0 > 0)
            def _():
                drain_store(g0 - 1, buf1, ssem1)

            fire_group(g1, buf1, gsem1)
            drain_group(buf0, gsem0)
            store_group(g0, buf0, ssem0)

            @pl.when(g1 + 1 < n_groups)
            def _():
                drain_store(g0, buf0, ssem0)
                fire_group(g1 + 1, buf0, gsem0)

            drain_group(buf1, gsem1)
            store_group(g1, buf1, ssem1)

            @pl.when(g1 + 1 >= n_groups)
            def _():
                drain_store(g0, buf0, ssem0)
                drain_store(g1, buf1, ssem1)

    out = gather_kernel(idx3, table)
    return out.reshape(batch, n_fields, dim)


# speedup vs baseline: 1.5758x; 1.0005x over previous
import functools

import jax
import jax.numpy as jnp
from jax import lax
from jax.experimental import pallas as pl
from jax.experimental.pallas import tpu as pltpu
from jax.experimental.pallas import tpu_sc as plsc

NUM_CORES = 2
NUM_SUBCORES = 16
NUM_WORKERS = NUM_CORES * NUM_SUBCORES
CHUNK = 1664
K = 1
GROUP = CHUNK * K


def kernel(indices, table):
    batch, n_fields = indices.shape
    n_rows, dim = table.shape
    total = batch * n_fields
    assert total % (NUM_WORKERS * GROUP) == 0
    per_worker = total // NUM_WORKERS
    n_chunks = per_worker // CHUNK
    n_groups = per_worker // GROUP
    assert n_groups % 2 == 0

    idx3 = indices.reshape(NUM_WORKERS, n_chunks, CHUNK)
    mesh = plsc.VectorSubcoreMesh(core_axis_name="c", subcore_axis_name="s")

    @functools.partial(
        pl.kernel,
        mesh=mesh,
        out_type=jax.ShapeDtypeStruct((total, dim), jnp.float32),
        scratch_types=[
            pltpu.VMEM((n_chunks, CHUNK), jnp.int32),
            pltpu.VMEM((GROUP, dim), jnp.float32),
            pltpu.VMEM((GROUP, dim), jnp.float32),
            pltpu.SemaphoreType.DMA,
            pltpu.SemaphoreType.DMA,
            pltpu.SemaphoreType.DMA,
            pltpu.SemaphoreType.DMA,
        ],
        compiler_params=pltpu.CompilerParams(use_tc_tiling_on_sc=False),
    )
    def gather_kernel(idx_hbm, table_hbm, out_hbm, idx_v, buf0, buf1,
                      gsem0, gsem1, ssem0, ssem1):
        wid = lax.axis_index("s") * NUM_CORES + lax.axis_index("c")
        base = wid * per_worker
        pltpu.sync_copy(idx_hbm.at[wid], idx_v)

        def fire_group(g, buf, gsem):
            for j in range(K):
                pltpu.async_copy(
                    table_hbm.at[idx_v.at[g * K + j]],
                    buf.at[pl.ds(j * CHUNK, CHUNK)],
                    gsem,
                )

        def drain_group(buf, gsem):
            pltpu.make_async_copy(
                table_hbm.at[pl.ds(0, GROUP)], buf, gsem
            ).wait()

        def store_group(g, buf, ssem):
            return pltpu.async_copy(
                buf, out_hbm.at[pl.ds(base + g * GROUP, GROUP)], ssem
            )

        def drain_store(g, buf, ssem):
            pltpu.make_async_copy(
                buf, out_hbm.at[pl.ds(base + g * GROUP, GROUP)], ssem
            ).wait()

        fire_group(0, buf0, gsem0)

        @pl.loop(0, n_groups, step=2)
        def _(i2):
            g0 = i2
            g1 = i2 + 1

            @pl.when(g0 > 0)
            def _():
                drain_store(g0 - 1, buf1, ssem1)

            fire_group(g1, buf1, gsem1)
            drain_group(buf0, gsem0)
            store_group(g0, buf0, ssem0)

            @pl.when(g1 + 1 < n_groups)
            def _():
                drain_store(g0, buf0, ssem0)
                fire_group(g1 + 1, buf0, gsem0)

            drain_group(buf1, gsem1)
            store_group(g1, buf1, ssem1)

            @pl.when(g1 + 1 >= n_groups)
            def _():
                drain_store(g0, buf0, ssem0)
                drain_store(g1, buf1, ssem1)

    out = gather_kernel(idx3, table)
    return out.reshape(batch, n_fields, dim)
